# baseline (speedup 1.0000x reference)
import functools

import jax
import jax.numpy as jnp
from jax import lax
from jax.experimental import pallas as pl
from jax.experimental.pallas import tpu as pltpu
from jax.experimental.pallas import tpu_sc as plsc

N = 10000
NP = 10240
E_TOT = 330000
E_PAD = 331776
K = 96
SUP = 1152
R = 2048
GRID = 5
INV_SQRT = 0.9999950000374997
NEG = -1e30
F32 = jnp.float32


def _t1_body(x_ref, w_ref, as_ref, ad_ref, h_ref, asrc_ref, adst_ref, m_ref,
             ms_ref, md_ref):
    i = pl.program_id(0)
    h = jnp.dot(x_ref[...], w_ref[...], preferred_element_type=F32)
    h0 = h[:, :128]
    h1 = h[:, 128:]
    s0 = jnp.sum(h0 * as_ref[0], axis=1)
    s1 = jnp.sum(h1 * as_ref[1], axis=1)
    d0 = jnp.sum(h0 * ad_ref[0], axis=1)
    d1 = jnp.sum(h1 * ad_ref[1], axis=1)
    h_ref[...] = jnp.stack([h0, h1])
    asrc_ref[...] = jnp.stack([s0, s1])
    adst_ref[...] = jnp.stack([d0, d1])

    valid = (i * R + lax.iota(jnp.int32, R)) < N

    @pl.when(i == 0)
    def _():
        ms_ref[0] = NEG
        ms_ref[1] = NEG
        md_ref[0] = NEG
        md_ref[1] = NEG

    ms_ref[0] = jnp.maximum(ms_ref[0], jnp.max(jnp.where(valid, s0, NEG)))
    ms_ref[1] = jnp.maximum(ms_ref[1], jnp.max(jnp.where(valid, s1, NEG)))
    md_ref[0] = jnp.maximum(md_ref[0], jnp.max(jnp.where(valid, d0, NEG)))
    md_ref[1] = jnp.maximum(md_ref[1], jnp.max(jnp.where(valid, d1, NEG)))
    b0 = ms_ref[0] + md_ref[0]
    b1 = ms_ref[1] + md_ref[1]
    m0 = jnp.maximum(b0, 0.2 * b0)
    m1 = jnp.maximum(b1, 0.2 * b1)
    m_ref[...] = jnp.stack([jnp.full((128,), m0, F32),
                            jnp.full((128,), m1, F32)])


def _t1(x, W1, a1s, a1d):
    return pl.pallas_call(
        _t1_body,
        grid=(GRID,),
        in_specs=[
            pl.BlockSpec((R, 128), lambda i: (i, 0)),
            pl.BlockSpec((128, 256), lambda i: (0, 0)),
            pl.BlockSpec((2, 128), lambda i: (0, 0)),
            pl.BlockSpec((2, 128), lambda i: (0, 0)),
        ],
        out_specs=[
            pl.BlockSpec((2, R, 128), lambda i: (0, i, 0)),
            pl.BlockSpec((2, R), lambda i: (0, i)),
            pl.BlockSpec((2, R), lambda i: (0, i)),
            pl.BlockSpec((2, 128), lambda i: (0, 0)),
        ],
        out_shape=[
            jax.ShapeDtypeStruct((2, N, 128), F32),
            jax.ShapeDtypeStruct((2, N), F32),
            jax.ShapeDtypeStruct((2, N), F32),
            jax.ShapeDtypeStruct((2, 128), F32),
        ],
        scratch_shapes=[pltpu.SMEM((2,), F32), pltpu.SMEM((2,), F32)],
    )(x, W1, a1s, a1d)


def _t23_body(o_ref, den_ref, b_ref, g_ref, be_ref, w_ref, as_ref, ad_ref,
              h_ref, asrc_ref, adst_ref, m_ref, ms_ref, md_ref, *, dout):
    i = pl.program_id(0)
    x0 = o_ref[0] / den_ref[0][:, None]
    x1 = o_ref[1] / den_ref[1][:, None]
    xcat = jnp.concatenate([x0, x1], axis=1) + b_ref[0]
    xx = jnp.maximum(g_ref[0] * (xcat * INV_SQRT) + be_ref[0], 0.0)
    h = jnp.dot(xx, w_ref[...], preferred_element_type=F32)
    s = jnp.sum(h * as_ref[0], axis=1)
    d = jnp.sum(h * ad_ref[0], axis=1)
    if dout == 256:
        h_ref[...] = jnp.stack([h[:, :128], h[:, 128:]])
    else:
        h_ref[...] = jnp.concatenate([h, jnp.zeros((R, 64 - dout), F32)],
                                     axis=1)[None]
    asrc_ref[...] = jnp.stack([s, s])
    adst_ref[...] = jnp.stack([d, d])

    valid = (i * R + lax.iota(jnp.int32, R)) < N

    @pl.when(i == 0)
    def _():
        ms_ref[0] = NEG
        md_ref[0] = NEG

    ms_ref[0] = jnp.maximum(ms_ref[0], jnp.max(jnp.where(valid, s, NEG)))
    md_ref[0] = jnp.maximum(md_ref[0], jnp.max(jnp.where(valid, d, NEG)))
    b0 = ms_ref[0] + md_ref[0]
    m0 = jnp.maximum(b0, 0.2 * b0)
    m_ref[...] = jnp.full((2, 128), m0, F32)


def _t23(o, den, b, g, be, W, a_s, a_d, dout):
    hshape = (2, N, 128) if dout == 256 else (1, N, 64)
    hblock = (2, R, 128) if dout == 256 else (1, R, 64)
    return pl.pallas_call(
        functools.partial(_t23_body, dout=dout),
        grid=(GRID,),
        in_specs=[
            pl.BlockSpec((2, R, 128), lambda i: (0, i, 0)),
            pl.BlockSpec((2, R), lambda i: (0, i)),
            pl.BlockSpec((1, 256), lambda i: (0, 0)),
            pl.BlockSpec((1, 256), lambda i: (0, 0)),
            pl.BlockSpec((1, 256), lambda i: (0, 0)),
            pl.BlockSpec((256, dout), lambda i: (0, 0)),
            pl.BlockSpec((1, dout), lambda i: (0, 0)),
            pl.BlockSpec((1, dout), lambda i: (0, 0)),
        ],
        out_specs=[
            pl.BlockSpec(hblock, lambda i: (0, i, 0)),
            pl.BlockSpec((2, R), lambda i: (0, i)),
            pl.BlockSpec((2, R), lambda i: (0, i)),
            pl.BlockSpec((2, 128), lambda i: (0, 0)),
        ],
        out_shape=[
            jax.ShapeDtypeStruct(hshape, F32),
            jax.ShapeDtypeStruct((2, N), F32),
            jax.ShapeDtypeStruct((2, N), F32),
            jax.ShapeDtypeStruct((2, 128), F32),
        ],
        scratch_shapes=[pltpu.SMEM((2,), F32), pltpu.SMEM((2,), F32)],
    )(o, den, b, g, be, W, a_s, a_d)


def _t4_body(o_ref, den_ref, b_ref, out_ref):
    o = o_ref[0] + o_ref[1]
    den = den_ref[0] + den_ref[1]
    logits = o[:, :40] / den[:, None] + b_ref[0]
    zmax = jnp.max(logits, axis=1, keepdims=True)
    z = logits - zmax
    out_ref[...] = z - jnp.log(jnp.sum(jnp.exp(z), axis=1, keepdims=True))


def _t4(o, den, b3):
    return pl.pallas_call(
        _t4_body,
        grid=(GRID,),
        in_specs=[
            pl.BlockSpec((2, R, 64), lambda i: (0, i, 0)),
            pl.BlockSpec((2, R), lambda i: (0, i)),
            pl.BlockSpec((1, 40), lambda i: (0, 0)),
        ],
        out_specs=pl.BlockSpec((R, 40), lambda i: (i, 0)),
        out_shape=jax.ShapeDtypeStruct((N, 40), F32),
    )(o, den, b3)


def _make_sc_pass(D, edge_split):
    per_tile = E_PAD // 32 if edge_split else E_PAD // 16
    nsup = per_tile // SUP
    t_off = 0 if edge_split else N
    mesh = plsc.VectorSubcoreMesh(core_axis_name="c", subcore_axis_name="s")

    n_chunks = SUP // K
    n_dbl = n_chunks // 2

    def body(h_hbm, asrc_hbm, adst_hbm, m_hbm, src_hbm, dst_hbm,
             out_hbm, den_hbm,
             src_ch, dst_ch, asrc_t, adst_t, m_t,
             src_a, dst_a, w_a, rows_a, src_b, dst_b, w_b, rows_b,
             acc, den_s, sem_a, sem_b):
        c = lax.axis_index("c")
        s = lax.axis_index("s")
        pltpu.sync_copy(asrc_hbm.at[pl.ds(c * N, N)], asrc_t)
        pltpu.sync_copy(adst_hbm.at[pl.ds(c * N, N)], adst_t)
        pltpu.sync_copy(m_hbm.at[pl.ds(c * 128, 128)], m_t)
        if edge_split:
            start = (c * 16 + s) * per_tile
        else:
            start = s * per_tile

        z = jnp.zeros((16,), F32)
        for j in range(K):
            for q in range(D // 16):
                rows_a[j, pl.ds(q * 16, 16)] = z
        for q in range(K // 16):
            w_a[pl.ds(q * 16, 16)] = z

        def zero_body(t, carry):
            pltpu.sync_copy(rows_a.at[pl.ds(0, 64)],
                            acc.at[pl.ds(s * 640 + t * 64, 64)])
            pltpu.sync_copy(w_a.at[pl.ds(0, 64)],
                            den_s.at[pl.ds(s * 640 + t * 64, 64)])
            return carry

        lax.fori_loop(0, 10, zero_body, 0)
        plsc.subcore_barrier()

        m_vec = m_t[pl.ds(0, 16)]
        iot = lax.iota(jnp.int32, 16)
        coff = c * t_off

        def issue(ci, sup_start, sbuf, dbuf, wbuf, rbuf, sem):
            for q in range(K // 16):
                sl = pl.ds(ci * K + q * 16, 16)
                sv = src_ch[sl]
                dv = dst_ch[sl]
                a1 = plsc.load_gather(asrc_t, [sv])
                a2 = plsc.load_gather(adst_t, [dv])
                t = a1 + a2
                e = jnp.maximum(t, 0.2 * t)
                w = jnp.exp(e - m_vec)
                eid = sup_start + ci * K + q * 16 + iot
                w = jnp.where(eid < E_TOT, w, 0.0)
                bsl = pl.ds(q * 16, 16)
                wbuf[bsl] = w
                sbuf[bsl] = sv + coff
                dbuf[bsl] = dv
            return pltpu.async_copy(h_hbm.at[sbuf], rbuf, sem)

        def process(dbuf, wbuf, rbuf):
            for q in range(K // 16):
                wq = wbuf[pl.ds(q * 16, 16)]
                for r in range(16):
                    j = q * 16 + r
                    wj = wq[r]
                    for p in range(D // 16):
                        sl = pl.ds(p * 16, 16)
                        rbuf[j, sl] = rbuf[j, sl] * wj
            pltpu.sync_copy(wbuf, den_s.at[dbuf], add=True)
            pltpu.sync_copy(rbuf, acc.at[dbuf], add=True)

        def super_body(g, carry):
            sup_start = start + g * SUP
            pltpu.sync_copy(src_hbm.at[pl.ds(sup_start, SUP)], src_ch)
            pltpu.sync_copy(dst_hbm.at[pl.ds(sup_start, SUP)], dst_ch)
            issue(0, sup_start, src_a, dst_a, w_a, rows_a, sem_a)
            issue(1, sup_start, src_b, dst_b, w_b, rows_b, sem_b)

            def dbl_body(i, carry2):
                pltpu.make_async_copy(h_hbm.at[src_a], rows_a, sem_a).wait()
                process(dst_a, w_a, rows_a)

                @pl.when(2 * i + 2 < n_chunks)
                def _():
                    issue(2 * i + 2, sup_start, src_a, dst_a, w_a, rows_a,
                          sem_a)
                pltpu.make_async_copy(h_hbm.at[src_b], rows_b, sem_b).wait()
                process(dst_b, w_b, rows_b)

                @pl.when(2 * i + 3 < n_chunks)
                def _():
                    issue(2 * i + 3, sup_start, src_b, dst_b, w_b, rows_b,
                          sem_b)
                return carry2

            lax.fori_loop(0, n_dbl, dbl_body, 0)
            return carry

        lax.fori_loop(0, nsup, super_body, 0)
        plsc.subcore_barrier()

        pltpu.sync_copy(acc.at[pl.ds(s * 640, 640)],
                        out_hbm.at[pl.ds(c * NP + s * 640, 640)])
        pltpu.sync_copy(den_s.at[pl.ds(s * 640, 640)],
                        den_hbm.at[pl.ds(c * NP + s * 640, 640)])

    return pl.kernel(
        body,
        out_type=[
            jax.ShapeDtypeStruct((2 * NP, D), F32),
            jax.ShapeDtypeStruct((2 * NP,), F32),
        ],
        mesh=mesh,
        compiler_params=pltpu.CompilerParams(needs_layout_passes=False,
                                             use_tc_tiling_on_sc=False),
        scratch_types=[
            pltpu.VMEM((SUP,), jnp.int32),
            pltpu.VMEM((SUP,), jnp.int32),
            pltpu.VMEM((N,), F32),
            pltpu.VMEM((N,), F32),
            pltpu.VMEM((128,), F32),
            pltpu.VMEM((K,), jnp.int32),
            pltpu.VMEM((K,), jnp.int32),
            pltpu.VMEM((K,), F32),
            pltpu.VMEM((K, D), F32),
            pltpu.VMEM((K,), jnp.int32),
            pltpu.VMEM((K,), jnp.int32),
            pltpu.VMEM((K,), F32),
            pltpu.VMEM((K, D), F32),
            pltpu.VMEM_SHARED((NP, D), F32),
            pltpu.VMEM_SHARED((NP,), F32),
            pltpu.SemaphoreType.DMA,
            pltpu.SemaphoreType.DMA,
        ],
    )


@functools.lru_cache(maxsize=None)
def _sc_pass(D, edge_split):
    return _make_sc_pass(D, edge_split)


def _sc_split(*args):
    return _sc_pass(128, False)(*args)


def _sc_edge(*args):
    return _sc_pass(64, True)(*args)


def kernel(x, edge_index, W1, a1s, a1d, b1, g1, be1, W2, a2s, a2d, b2, g2,
           be2, W3, a3s, a3d, b3):
    loop = jnp.arange(N, dtype=jnp.int32)
    zpad = jnp.zeros((E_PAD - E_TOT,), jnp.int32)
    src = jnp.concatenate([edge_index[0].astype(jnp.int32), loop, zpad])
    dst = jnp.concatenate([edge_index[1].astype(jnp.int32), loop, zpad])

    H1, asrc1, adst1, m1 = _t1(x, W1, a1s, a1d)
    o1, den1 = _sc_split(H1.reshape(2 * N, 128), asrc1.reshape(2 * N),
                         adst1.reshape(2 * N), m1.reshape(256), src, dst)
    o1 = o1.reshape(2, NP, 128)
    den1 = den1.reshape(2, NP)

    H2, asrc2, adst2, m2 = _t23(o1, den1, b1.reshape(1, 256),
                                g1.reshape(1, 256), be1.reshape(1, 256),
                                W2, a2s, a2d, 256)
    o2, den2 = _sc_split(H2.reshape(2 * N, 128), asrc2.reshape(2 * N),
                         adst2.reshape(2 * N), m2.reshape(256), src, dst)
    o2 = o2.reshape(2, NP, 128)
    den2 = den2.reshape(2, NP)

    H3, asrc3, adst3, m3 = _t23(o2, den2, b2.reshape(1, 256),
                                g2.reshape(1, 256), be2.reshape(1, 256),
                                W3, a3s, a3d, 40)
    o3, den3 = _sc_edge(H3.reshape(N, 64), asrc3.reshape(2 * N),
                        adst3.reshape(2 * N), m3.reshape(256), src, dst)
    o3 = o3.reshape(2, NP, 64)
    den3 = den3.reshape(2, NP)

    return _t4(o3, den3, b3.reshape(1, 40))


# speedup vs baseline: 1.1348x; 1.1348x over previous
import functools

import jax
import jax.numpy as jnp
from jax import lax
from jax.experimental import pallas as pl
from jax.experimental.pallas import tpu as pltpu
from jax.experimental.pallas import tpu_sc as plsc

N = 10000
NP = 10240
E_TOT = 330000
E_PAD = 331776
K = 64
SUP = 1152
R = 2048
GRID = 5
INV_SQRT = 0.9999950000374997
NEG = -1e30
F32 = jnp.float32


def _t1_body(x_ref, w_ref, as_ref, ad_ref, h_ref, asrc_ref, adst_ref, m_ref,
             ms_ref, md_ref):
    i = pl.program_id(0)
    h = jnp.dot(x_ref[...], w_ref[...], preferred_element_type=F32)
    h0 = h[:, :128]
    h1 = h[:, 128:]
    s0 = jnp.sum(h0 * as_ref[0], axis=1)
    s1 = jnp.sum(h1 * as_ref[1], axis=1)
    d0 = jnp.sum(h0 * ad_ref[0], axis=1)
    d1 = jnp.sum(h1 * ad_ref[1], axis=1)
    h_ref[...] = jnp.stack([h0, h1])
    asrc_ref[...] = jnp.stack([s0, s1])
    adst_ref[...] = jnp.stack([d0, d1])

    valid = (i * R + lax.iota(jnp.int32, R)) < N

    @pl.when(i == 0)
    def _():
        ms_ref[0] = NEG
        ms_ref[1] = NEG
        md_ref[0] = NEG
        md_ref[1] = NEG

    ms_ref[0] = jnp.maximum(ms_ref[0], jnp.max(jnp.where(valid, s0, NEG)))
    ms_ref[1] = jnp.maximum(ms_ref[1], jnp.max(jnp.where(valid, s1, NEG)))
    md_ref[0] = jnp.maximum(md_ref[0], jnp.max(jnp.where(valid, d0, NEG)))
    md_ref[1] = jnp.maximum(md_ref[1], jnp.max(jnp.where(valid, d1, NEG)))
    b0 = ms_ref[0] + md_ref[0]
    b1 = ms_ref[1] + md_ref[1]
    m0 = jnp.maximum(b0, 0.2 * b0)
    m1 = jnp.maximum(b1, 0.2 * b1)
    m_ref[...] = jnp.stack([jnp.full((128,), m0, F32),
                            jnp.full((128,), m1, F32)])


def _t1(x, W1, a1s, a1d):
    return pl.pallas_call(
        _t1_body,
        grid=(GRID,),
        in_specs=[
            pl.BlockSpec((R, 128), lambda i: (i, 0)),
            pl.BlockSpec((128, 256), lambda i: (0, 0)),
            pl.BlockSpec((2, 128), lambda i: (0, 0)),
            pl.BlockSpec((2, 128), lambda i: (0, 0)),
        ],
        out_specs=[
            pl.BlockSpec((2, R, 128), lambda i: (0, i, 0)),
            pl.BlockSpec((2, R), lambda i: (0, i)),
            pl.BlockSpec((2, R), lambda i: (0, i)),
            pl.BlockSpec((2, 128), lambda i: (0, 0)),
        ],
        out_shape=[
            jax.ShapeDtypeStruct((2, N, 128), F32),
            jax.ShapeDtypeStruct((2, N), F32),
            jax.ShapeDtypeStruct((2, N), F32),
            jax.ShapeDtypeStruct((2, 128), F32),
        ],
        scratch_shapes=[pltpu.SMEM((2,), F32), pltpu.SMEM((2,), F32)],
    )(x, W1, a1s, a1d)


def _t23_body(o_ref, den_ref, b_ref, g_ref, be_ref, w_ref, as_ref, ad_ref,
              h_ref, asrc_ref, adst_ref, m_ref, ms_ref, md_ref, *, dout):
    i = pl.program_id(0)
    x0 = o_ref[0] / den_ref[0][:, None]
    x1 = o_ref[1] / den_ref[1][:, None]
    xcat = jnp.concatenate([x0, x1], axis=1) + b_ref[0]
    xx = jnp.maximum(g_ref[0] * (xcat * INV_SQRT) + be_ref[0], 0.0)
    h = jnp.dot(xx, w_ref[...], preferred_element_type=F32)
    s = jnp.sum(h * as_ref[0], axis=1)
    d = jnp.sum(h * ad_ref[0], axis=1)
    if dout == 256:
        h_ref[...] = jnp.stack([h[:, :128], h[:, 128:]])
    else:
        h_ref[...] = jnp.concatenate([h, jnp.zeros((R, 64 - dout), F32)],
                                     axis=1)[None]
    asrc_ref[...] = jnp.stack([s, s])
    adst_ref[...] = jnp.stack([d, d])

    valid = (i * R + lax.iota(jnp.int32, R)) < N

    @pl.when(i == 0)
    def _():
        ms_ref[0] = NEG
        md_ref[0] = NEG

    ms_ref[0] = jnp.maximum(ms_ref[0], jnp.max(jnp.where(valid, s, NEG)))
    md_ref[0] = jnp.maximum(md_ref[0], jnp.max(jnp.where(valid, d, NEG)))
    b0 = ms_ref[0] + md_ref[0]
    m0 = jnp.maximum(b0, 0.2 * b0)
    m_ref[...] = jnp.full((2, 128), m0, F32)


def _t23(o, den, b, g, be, W, a_s, a_d, dout):
    hshape = (2, N, 128) if dout == 256 else (1, N, 64)
    hblock = (2, R, 128) if dout == 256 else (1, R, 64)
    return pl.pallas_call(
        functools.partial(_t23_body, dout=dout),
        grid=(GRID,),
        in_specs=[
            pl.BlockSpec((2, R, 128), lambda i: (0, i, 0)),
            pl.BlockSpec((2, R), lambda i: (0, i)),
            pl.BlockSpec((1, 256), lambda i: (0, 0)),
            pl.BlockSpec((1, 256), lambda i: (0, 0)),
            pl.BlockSpec((1, 256), lambda i: (0, 0)),
            pl.BlockSpec((256, dout), lambda i: (0, 0)),
            pl.BlockSpec((1, dout), lambda i: (0, 0)),
            pl.BlockSpec((1, dout), lambda i: (0, 0)),
        ],
        out_specs=[
            pl.BlockSpec(hblock, lambda i: (0, i, 0)),
            pl.BlockSpec((2, R), lambda i: (0, i)),
            pl.BlockSpec((2, R), lambda i: (0, i)),
            pl.BlockSpec((2, 128), lambda i: (0, 0)),
        ],
        out_shape=[
            jax.ShapeDtypeStruct(hshape, F32),
            jax.ShapeDtypeStruct((2, N), F32),
            jax.ShapeDtypeStruct((2, N), F32),
            jax.ShapeDtypeStruct((2, 128), F32),
        ],
        scratch_shapes=[pltpu.SMEM((2,), F32), pltpu.SMEM((2,), F32)],
    )(o, den, b, g, be, W, a_s, a_d)


def _t4_body(o_ref, den_ref, b_ref, out_ref):
    o = o_ref[0] + o_ref[1]
    den = den_ref[0] + den_ref[1]
    logits = o[:, :40] / den[:, None] + b_ref[0]
    zmax = jnp.max(logits, axis=1, keepdims=True)
    z = logits - zmax
    out_ref[...] = z - jnp.log(jnp.sum(jnp.exp(z), axis=1, keepdims=True))


def _t4(o, den, b3):
    return pl.pallas_call(
        _t4_body,
        grid=(GRID,),
        in_specs=[
            pl.BlockSpec((2, R, 64), lambda i: (0, i, 0)),
            pl.BlockSpec((2, R), lambda i: (0, i)),
            pl.BlockSpec((1, 40), lambda i: (0, 0)),
        ],
        out_specs=pl.BlockSpec((R, 40), lambda i: (i, 0)),
        out_shape=jax.ShapeDtypeStruct((N, 40), F32),
    )(o, den, b3)


def _make_sc_pass(D, edge_split):
    per_tile = E_PAD // 32 if edge_split else E_PAD // 16
    nsup = per_tile // SUP
    t_off = 0 if edge_split else N
    mesh = plsc.VectorSubcoreMesh(core_axis_name="c", subcore_axis_name="s")

    n_chunks = SUP // K
    n_dbl = n_chunks // 2

    def body(h_hbm, asrc_hbm, adst_hbm, m_hbm, src_hbm, dst_hbm,
             out_hbm, den_hbm,
             src_ch, dst_ch, asrc_t, adst_t, m_t,
             src_a, dst_a, w_a, rows_a, src_b, dst_b, w_b, rows_b,
             acc, den_s, sem_a, sem_b):
        c = lax.axis_index("c")
        s = lax.axis_index("s")
        pltpu.sync_copy(asrc_hbm.at[pl.ds(c * N, N)], asrc_t)
        pltpu.sync_copy(adst_hbm.at[pl.ds(c * N, N)], adst_t)
        pltpu.sync_copy(m_hbm.at[pl.ds(c * 128, 128)], m_t)
        if edge_split:
            start = (c * 16 + s) * per_tile
        else:
            start = s * per_tile

        z = jnp.zeros((16,), F32)
        for j in range(K):
            for q in range(D // 16):
                rows_a[j, pl.ds(q * 16, 16)] = z
        for q in range(K // 16):
            w_a[pl.ds(q * 16, 16)] = z

        def zero_body(t, carry):
            pltpu.sync_copy(rows_a, acc.at[pl.ds(s * 640 + t * K, K)])
            pltpu.sync_copy(w_a, den_s.at[pl.ds(s * 640 + t * K, K)])
            return carry

        lax.fori_loop(0, 640 // K, zero_body, 0)
        plsc.subcore_barrier()

        m_vec = m_t[pl.ds(0, 16)]
        iot = lax.iota(jnp.int32, 16)
        coff = c * t_off

        def issue(ci, sup_start, sbuf, dbuf, wbuf, rbuf, sem):
            for q in range(K // 16):
                sl = pl.ds(ci * K + q * 16, 16)
                sv = src_ch[sl]
                dv = dst_ch[sl]
                a1 = plsc.load_gather(asrc_t, [sv])
                a2 = plsc.load_gather(adst_t, [dv])
                t = a1 + a2
                e = jnp.maximum(t, 0.2 * t)
                w = jnp.exp(e - m_vec)
                eid = sup_start + ci * K + q * 16 + iot
                w = jnp.where(eid < E_TOT, w, 0.0)
                bsl = pl.ds(q * 16, 16)
                wbuf[bsl] = w
                sbuf[bsl] = sv + coff
                dbuf[bsl] = dv
            return pltpu.async_copy(h_hbm.at[sbuf], rbuf, sem)

        def process(dbuf, wbuf, rbuf):
            for q in range(K // 16):
                wq = wbuf[pl.ds(q * 16, 16)]
                for r in range(16):
                    j = q * 16 + r
                    wj = wq[r]
                    for p in range(D // 16):
                        sl = pl.ds(p * 16, 16)
                        rbuf[j, sl] = rbuf[j, sl] * wj
            pltpu.sync_copy(wbuf, den_s.at[dbuf], add=True)
            pltpu.sync_copy(rbuf, acc.at[dbuf], add=True)

        def super_body(g, carry):
            sup_start = start + g * SUP
            pltpu.sync_copy(src_hbm.at[pl.ds(sup_start, SUP)], src_ch)
            pltpu.sync_copy(dst_hbm.at[pl.ds(sup_start, SUP)], dst_ch)
            issue(0, sup_start, src_a, dst_a, w_a, rows_a, sem_a)
            issue(1, sup_start, src_b, dst_b, w_b, rows_b, sem_b)

            def dbl_body(i, carry2):
                pltpu.make_async_copy(h_hbm.at[src_a], rows_a, sem_a).wait()
                process(dst_a, w_a, rows_a)

                @pl.when(2 * i + 2 < n_chunks)
                def _():
                    issue(2 * i + 2, sup_start, src_a, dst_a, w_a, rows_a,
                          sem_a)
                pltpu.make_async_copy(h_hbm.at[src_b], rows_b, sem_b).wait()
                process(dst_b, w_b, rows_b)

                @pl.when(2 * i + 3 < n_chunks)
                def _():
                    issue(2 * i + 3, sup_start, src_b, dst_b, w_b, rows_b,
                          sem_b)
                return carry2

            lax.fori_loop(0, n_dbl, dbl_body, 0)
            return carry

        lax.fori_loop(0, nsup, super_body, 0)
        plsc.subcore_barrier()

        pltpu.sync_copy(acc.at[pl.ds(s * 640, 640)],
                        out_hbm.at[pl.ds(c * NP + s * 640, 640)])
        pltpu.sync_copy(den_s.at[pl.ds(s * 640, 640)],
                        den_hbm.at[pl.ds(c * NP + s * 640, 640)])

    return pl.kernel(
        body,
        out_type=[
            jax.ShapeDtypeStruct((2 * NP, D), F32),
            jax.ShapeDtypeStruct((2 * NP,), F32),
        ],
        mesh=mesh,
        compiler_params=pltpu.CompilerParams(needs_layout_passes=False,
                                             use_tc_tiling_on_sc=False),
        scratch_types=[
            pltpu.VMEM((SUP,), jnp.int32),
            pltpu.VMEM((SUP,), jnp.int32),
            pltpu.VMEM((N,), F32),
            pltpu.VMEM((N,), F32),
            pltpu.VMEM((128,), F32),
            pltpu.VMEM((K,), jnp.int32),
            pltpu.VMEM((K,), jnp.int32),
            pltpu.VMEM((K,), F32),
            pltpu.VMEM((K, D), F32),
            pltpu.VMEM((K,), jnp.int32),
            pltpu.VMEM((K,), jnp.int32),
            pltpu.VMEM((K,), F32),
            pltpu.VMEM((K, D), F32),
            pltpu.VMEM_SHARED((NP, D), F32),
            pltpu.VMEM_SHARED((NP,), F32),
            pltpu.SemaphoreType.DMA,
            pltpu.SemaphoreType.DMA,
        ],
    )


@functools.lru_cache(maxsize=None)
def _sc_pass(D, edge_split):
    return _make_sc_pass(D, edge_split)


def _sc_split(*args):
    return _sc_pass(128, False)(*args)


def _sc_edge(*args):
    return _sc_pass(64, True)(*args)


def kernel(x, edge_index, W1, a1s, a1d, b1, g1, be1, W2, a2s, a2d, b2, g2,
           be2, W3, a3s, a3d, b3):
    loop = jnp.arange(N, dtype=jnp.int32)
    zpad = jnp.zeros((E_PAD - E_TOT,), jnp.int32)
    src = jnp.concatenate([edge_index[0].astype(jnp.int32), loop, zpad])
    dst = jnp.concatenate([edge_index[1].astype(jnp.int32), loop, zpad])

    H1, asrc1, adst1, m1 = _t1(x, W1, a1s, a1d)
    o1, den1 = _sc_split(H1.reshape(2 * N, 128), asrc1.reshape(2 * N),
                         adst1.reshape(2 * N), m1.reshape(256), src, dst)
    o1 = o1.reshape(2, NP, 128)
    den1 = den1.reshape(2, NP)

    H2, asrc2, adst2, m2 = _t23(o1, den1, b1.reshape(1, 256),
                                g1.reshape(1, 256), be1.reshape(1, 256),
                                W2, a2s, a2d, 256)
    o2, den2 = _sc_split(H2.reshape(2 * N, 128), asrc2.reshape(2 * N),
                         adst2.reshape(2 * N), m2.reshape(256), src, dst)
    o2 = o2.reshape(2, NP, 128)
    den2 = den2.reshape(2, NP)

    H3, asrc3, adst3, m3 = _t23(o2, den2, b2.reshape(1, 256),
                                g2.reshape(1, 256), be2.reshape(1, 256),
                                W3, a3s, a3d, 40)
    o3, den3 = _sc_edge(H3.reshape(N, 64), asrc3.reshape(2 * N),
                        adst3.reshape(2 * N), m3.reshape(256), src, dst)
    o3 = o3.reshape(2, NP, 64)
    den3 = den3.reshape(2, NP)

    return _t4(o3, den3, b3.reshape(1, 40))
